# baseline (speedup 1.0000x reference)
import functools

import jax
import jax.numpy as jnp
from jax import lax
from jax.experimental import pallas as pl
from jax.experimental.pallas import tpu as pltpu
from jax.experimental.pallas import tpu_sc as plsc

N = 10000
R2 = 10000
D = 128
E = 160000
T = 3
P = 2 * E
TBL = N * R2

NC = 2
NS = 16
NW = NC * NS

NPAD = 10240
DUMMY = 10000
RPT = NPAD // NS

CA = 80
PWA = P // NW
NCHA = PWA // CA
PHA = 40
PHASES_A = ((0, 40), (40, 40), (80, 40), (120, 5))

CB = 64
NCHB = 80
PHB = 40
EPAD = NW * NCHB * CB

TCB = 1000


@functools.lru_cache(maxsize=None)
def _mesh():
    return plsc.VectorSubcoreMesh(core_axis_name="c", subcore_axis_name="s")


def _wid():
    return lax.axis_index("c") * NS + lax.axis_index("s")


def _zero_fill_1d(zvec):
    n = zvec.shape[0]

    @pl.loop(0, n // 16)
    def _(i):
        zvec[pl.ds(i * 16, 16)] = jnp.zeros((16,), jnp.float32)


@functools.lru_cache(maxsize=None)
def _make_a1():
    @functools.partial(
        pl.kernel,
        out_type=(
            pltpu.HBM((TBL,), jnp.int32),
            pltpu.HBM((NPAD,), jnp.float32),
        ),
        mesh=_mesh(),
        scratch_types=[
            pltpu.VMEM((NCHA, CA), jnp.int32),
            pltpu.VMEM((NCHA, CA), jnp.int32),
            pltpu.VMEM((NCHA, CA), jnp.int32),
            pltpu.VMEM((NCHA, CA), jnp.int32),
            pltpu.VMEM((1, CA), jnp.float32),
            pltpu.VMEM((RPT,), jnp.float32),
            pltpu.VMEM_SHARED((NPAD,), jnp.float32),
            pltpu.SemaphoreType.DMA((2,)),
            pltpu.SemaphoreType.DMA((2,)),
            pltpu.SemaphoreType.DMA((2,)),
        ],
    )
    def a1(ents, rels, table, deg_out,
           entbig, relbig, pidbig, valbig, onesb, zvec, deg_sh,
           semi, sems, semd):
        cid = lax.axis_index("c")
        sid = lax.axis_index("s")
        wid = _wid()
        base = wid * PWA
        r0 = sid * RPT
        on_core1 = cid == 1

        pltpu.async_copy(ents.at[wid], entbig, semi.at[0])
        pltpu.async_copy(rels.at[wid], relbig, semi.at[1])

        _zero_fill_1d(zvec)
        for m in range(CA // 16):
            onesb[0, pl.ds(m * 16, 16)] = jnp.ones((16,), jnp.float32)

        @pl.when(on_core1)
        def _():
            pltpu.sync_copy(zvec, deg_sh.at[pl.ds(r0, RPT)])

        plsc.subcore_barrier()

        pltpu.make_async_copy(ents.at[wid], entbig, semi.at[0]).wait()
        pltpu.make_async_copy(rels.at[wid], relbig, semi.at[1]).wait()

        def step(j, b):
            for m in range(CA // 16):
                sl = pl.ds(m * 16, 16)
                pidbig[j, sl] = entbig[j, sl] * R2 + relbig[j, sl]
                valbig[j, sl] = (base + j * CA + m * 16) + lax.iota(jnp.int32, 16)
            pltpu.async_copy(valbig.at[j], table.at[pidbig.at[j]], sems.at[b])

            @pl.when(on_core1)
            def _():
                pltpu.async_copy(onesb.at[0], deg_sh.at[entbig.at[j]],
                                 semd.at[b], add=True)

        @pl.loop(0, NCHA - 1, step=2)
        def _(j0):
            step(j0, 0)
            step(j0 + 1, 1)

        step(NCHA - 1, 0)

        @pl.loop(0, (NCHA + 1) // 2)
        def _(j):
            pltpu.make_async_copy(valbig.at[0], table.at[pidbig.at[0]], sems.at[0]).wait()

        @pl.loop(0, NCHA // 2)
        def _(j):
            pltpu.make_async_copy(valbig.at[0], table.at[pidbig.at[0]], sems.at[1]).wait()

        @pl.when(on_core1)
        def _():
            @pl.loop(0, (NCHA + 1) // 2)
            def _(j):
                pltpu.make_async_copy(onesb.at[0], deg_sh.at[entbig.at[0]], semd.at[0]).wait()

            @pl.loop(0, NCHA // 2)
            def _(j):
                pltpu.make_async_copy(onesb.at[0], deg_sh.at[entbig.at[0]], semd.at[1]).wait()

            plsc.subcore_barrier()
            pltpu.sync_copy(deg_sh.at[pl.ds(r0, RPT)], deg_out.at[pl.ds(r0, RPT)])

    return a1


@functools.lru_cache(maxsize=None)
def _make_a2():
    @functools.partial(
        pl.kernel,
        out_type=(
            pltpu.HBM((NC, NPAD, D), jnp.float32),
            pltpu.HBM((NC, NPAD), jnp.float32),
        ),
        mesh=_mesh(),
        scratch_types=[
            pltpu.VMEM((PHA, CA), jnp.int32),
            pltpu.VMEM((PHA, CA), jnp.int32),
            pltpu.VMEM((2, CA), jnp.int32),
            pltpu.VMEM((2, CA), jnp.int32),
            pltpu.VMEM((2, CA), jnp.int32),
            pltpu.VMEM((2, CA, D), jnp.float32),
            pltpu.VMEM((1, CA), jnp.float32),
            pltpu.VMEM((RPT,), jnp.float32),
            pltpu.VMEM_SHARED((NPAD, D), jnp.float32),
            pltpu.VMEM_SHARED((NPAD,), jnp.float32),
            pltpu.SemaphoreType.DMA((2,)),
            pltpu.SemaphoreType.DMA((2,)),
            pltpu.SemaphoreType.DMA((2,)),
        ],
    )
    def a2(ents, rels, table, h, sums_out, cnt_out,
           entbig, relbig, pidb, tvb, selb, rowsb, onesb, zvec,
           sums_sh, cnt_sh, semi, semg, semsc):
        cid = lax.axis_index("c")
        sid = lax.axis_index("s")
        wid = _wid()
        base = wid * PWA
        r0 = sid * RPT

        def load_idx(p0, ph):
            pltpu.async_copy(ents.at[wid, pl.ds(p0, ph)], entbig.at[pl.ds(0, ph)], semi.at[0])
            pltpu.async_copy(rels.at[wid, pl.ds(p0, ph)], relbig.at[pl.ds(0, ph)], semi.at[1])

        def wait_idx(p0, ph):
            pltpu.make_async_copy(ents.at[wid, pl.ds(p0, ph)], entbig.at[pl.ds(0, ph)], semi.at[0]).wait()
            pltpu.make_async_copy(rels.at[wid, pl.ds(p0, ph)], relbig.at[pl.ds(0, ph)], semi.at[1]).wait()

        load_idx(0, PHA)

        @pl.loop(0, CA)
        def _(i):
            for k in range(D // 16):
                rowsb[0, i, pl.ds(k * 16, 16)] = jnp.zeros((16,), jnp.float32)

        _zero_fill_1d(zvec)
        for m in range(CA // 16):
            onesb[0, pl.ds(m * 16, 16)] = jnp.ones((16,), jnp.float32)

        @pl.loop(0, RPT // CA)
        def _(jj):
            pltpu.sync_copy(rowsb.at[0], sums_sh.at[pl.ds(r0 + jj * CA, CA)])

        pltpu.sync_copy(zvec, cnt_sh.at[pl.ds(r0, RPT)])
        plsc.subcore_barrier()

        def fire_gather(i, sl):
            for m in range(CA // 16):
                slx = pl.ds(m * 16, 16)
                pidb[sl, slx] = entbig[i, slx] * R2 + relbig[i, slx]
            pltpu.async_copy(table.at[pidb.at[sl]], tvb.at[sl], semg.at[sl])
            pltpu.async_copy(h.at[entbig.at[i]], rowsb.at[sl], semg.at[sl])

        def wait_scatters(sl):
            pltpu.make_async_copy(rowsb.at[sl], sums_sh.at[selb.at[sl]], semsc.at[sl]).wait()
            pltpu.make_async_copy(onesb.at[0], cnt_sh.at[selb.at[sl]], semsc.at[sl]).wait()

        def consume(i, p0, sl):
            pltpu.make_async_copy(table.at[pidb.at[sl]], tvb.at[sl], semg.at[sl]).wait()
            pltpu.make_async_copy(h.at[entbig.at[0]], rowsb.at[sl], semg.at[sl]).wait()
            for m in range(CA // 16):
                slx = pl.ds(m * 16, 16)
                val16 = (base + p0 * CA + i * CA + m * 16) + lax.iota(jnp.int32, 16)
                first = tvb[sl, slx] == val16
                selb[sl, slx] = jnp.where(first, relbig[i, slx], DUMMY)
            pltpu.async_copy(rowsb.at[sl], sums_sh.at[selb.at[sl]], semsc.at[sl], add=True)
            pltpu.async_copy(onesb.at[0], cnt_sh.at[selb.at[sl]], semsc.at[sl], add=True)

        for pi, (p0, ph) in enumerate(PHASES_A):
            if pi > 0:
                prev_ph = PHASES_A[pi - 1][1]
                wait_scatters((prev_ph - 1) % 2)
                load_idx(p0, ph)
                wait_idx(p0, ph)
            else:
                wait_idx(0, PHA)
            fire_gather(0, 0)
            if ph % 2 == 0:
                @pl.loop(0, ph - 2, step=2)
                def _(j0, p0=p0):
                    @pl.when(j0 >= 1)
                    def _():
                        wait_scatters(1)

                    fire_gather(j0 + 1, 1)
                    consume(j0, p0, 0)
                    wait_scatters(0)
                    fire_gather(j0 + 2, 0)
                    consume(j0 + 1, p0, 1)

                if ph > 2:
                    wait_scatters(1)
                fire_gather(ph - 1, 1)
                consume(ph - 2, p0, 0)
                wait_scatters(0)
                consume(ph - 1, p0, 1)
            else:
                @pl.loop(0, ph - 1, step=2)
                def _(j0, p0=p0):
                    @pl.when(j0 >= 1)
                    def _():
                        wait_scatters(1)

                    fire_gather(j0 + 1, 1)
                    consume(j0, p0, 0)
                    wait_scatters(0)
                    fire_gather(j0 + 2, 0)
                    consume(j0 + 1, p0, 1)

                if ph > 2:
                    wait_scatters(1)
                consume(ph - 1, p0, 0)

        wait_scatters((PHASES_A[-1][1] - 1) % 2)
        plsc.subcore_barrier()

        @pl.loop(0, RPT // 128)
        def _(jj):
            pltpu.sync_copy(sums_sh.at[pl.ds(r0 + jj * 128, 128)],
                            sums_out.at[cid, pl.ds(r0 + jj * 128, 128)])

        pltpu.sync_copy(cnt_sh.at[pl.ds(r0, RPT)], cnt_out.at[cid, pl.ds(r0, RPT)])

    return a2


@functools.lru_cache(maxsize=None)
def _make_b():
    @functools.partial(
        pl.kernel,
        out_type=pltpu.HBM((NC, NPAD, D), jnp.float32),
        mesh=_mesh(),
        scratch_types=[
            pltpu.VMEM((PHB, CB), jnp.int32),
            pltpu.VMEM((PHB, CB), jnp.int32),
            pltpu.VMEM((PHB, CB), jnp.int32),
            pltpu.VMEM((2, CB, D), jnp.float32),
            pltpu.VMEM((2, CB, D), jnp.float32),
            pltpu.VMEM_SHARED((NPAD, D), jnp.float32),
            pltpu.SemaphoreType.DMA((3,)),
            pltpu.SemaphoreType.DMA((2,)),
            pltpu.SemaphoreType.DMA((2,)),
        ],
    )
    def b(src, rel, dst, taba, tabb, agg_out,
          sb, rb, db, rowsa, rowsb, agg_sh, semi, semg, semsc):
        cid = lax.axis_index("c")
        sid = lax.axis_index("s")
        wid = _wid()
        r0 = sid * RPT

        def load_idx(p0):
            pltpu.async_copy(src.at[wid, pl.ds(p0, PHB)], sb, semi.at[0])
            pltpu.async_copy(rel.at[wid, pl.ds(p0, PHB)], rb, semi.at[1])
            pltpu.async_copy(dst.at[wid, pl.ds(p0, PHB)], db, semi.at[2])

        def wait_idx(p0):
            pltpu.make_async_copy(src.at[wid, pl.ds(p0, PHB)], sb, semi.at[0]).wait()
            pltpu.make_async_copy(rel.at[wid, pl.ds(p0, PHB)], rb, semi.at[1]).wait()
            pltpu.make_async_copy(dst.at[wid, pl.ds(p0, PHB)], db, semi.at[2]).wait()

        load_idx(0)

        @pl.loop(0, CB)
        def _(i):
            for k in range(D // 16):
                rowsa[0, i, pl.ds(k * 16, 16)] = jnp.zeros((16,), jnp.float32)

        @pl.loop(0, RPT // CB)
        def _(jj):
            pltpu.sync_copy(rowsa.at[0], agg_sh.at[pl.ds(r0 + jj * CB, CB)])

        plsc.subcore_barrier()

        def fire_gather(i, sl):
            pltpu.async_copy(taba.at[sb.at[i]], rowsa.at[sl], semg.at[sl])
            pltpu.async_copy(tabb.at[rb.at[i]], rowsb.at[sl], semg.at[sl])

        def wait_scatters(sl):
            pltpu.make_async_copy(rowsa.at[sl], agg_sh.at[db.at[0]], semsc.at[sl]).wait()

        def consume(i, sl):
            pltpu.make_async_copy(taba.at[sb.at[0]], rowsa.at[sl], semg.at[sl]).wait()
            pltpu.make_async_copy(tabb.at[rb.at[0]], rowsb.at[sl], semg.at[sl]).wait()
            @pl.loop(0, CB)
            def _(r):
                for k in range(D // 16):
                    slx = pl.ds(k * 16, 16)
                    rowsa[sl, r, slx] = rowsa[sl, r, slx] + rowsb[sl, r, slx]

            pltpu.async_copy(rowsa.at[sl], agg_sh.at[db.at[i]], semsc.at[sl], add=True)

        for ph in range(NCHB // PHB):
            if ph > 0:
                wait_scatters(1)
                load_idx(ph * PHB)
                wait_idx(ph * PHB)
            else:
                wait_idx(0)
            fire_gather(0, 0)

            @pl.loop(0, PHB - 2, step=2)
            def _(j0):
                @pl.when(j0 >= 1)
                def _():
                    wait_scatters(1)

                fire_gather(j0 + 1, 1)
                consume(j0, 0)
                wait_scatters(0)
                fire_gather(j0 + 2, 0)
                consume(j0 + 1, 1)

            wait_scatters(1)
            fire_gather(PHB - 1, 1)
            consume(PHB - 2, 0)
            wait_scatters(0)
            consume(PHB - 1, 1)

        wait_scatters(1)
        plsc.subcore_barrier()

        @pl.loop(0, RPT // 128)
        def _(jj):
            pltpu.sync_copy(agg_sh.at[pl.ds(r0 + jj * 128, 128)],
                            agg_out.at[cid, pl.ds(r0 + jj * 128, 128)])

    return b


def _tc_norm(x):
    def body(x_ref, o_ref):
        v = x_ref[...]
        n = jnp.sqrt(jnp.sum(v * v, axis=1, keepdims=True))
        o_ref[...] = v / jnp.maximum(n, 1e-12)

    return pl.pallas_call(
        body,
        out_shape=jax.ShapeDtypeStruct((N, D), jnp.float32),
        grid=(N // TCB,),
        in_specs=[pl.BlockSpec((TCB, D), lambda i: (i, 0))],
        out_specs=pl.BlockSpec((TCB, D), lambda i: (i, 0)),
    )(x)


def _tc_rel(sums2, counts3, srel, hidden, wihs, wihd, whh, bih, bhh):
    def body(s_ref, c_ref, sr_ref, hid_ref, wihs_ref, wihd_ref, whh_ref,
             bih_ref, bhh_ref, o_ref):
        s = s_ref[0] + s_ref[1]
        c = c_ref[0] + c_ref[1]
        dyn = jnp.where(c > 0.0, s / jnp.maximum(c, 1.0), 0.0)
        gi = (jnp.dot(sr_ref[...], wihs_ref[...], preferred_element_type=jnp.float32)
              + jnp.dot(dyn, wihd_ref[...], preferred_element_type=jnp.float32)
              + bih_ref[...])
        gh = (jnp.dot(hid_ref[...], whh_ref[...], preferred_element_type=jnp.float32)
              + bhh_ref[...])
        rg = jax.nn.sigmoid(gi[:, :D] + gh[:, :D])
        zg = jax.nn.sigmoid(gi[:, D:2 * D] + gh[:, D:2 * D])
        ng = jnp.tanh(gi[:, 2 * D:] + rg * gh[:, 2 * D:])
        o_ref[...] = (1.0 - zg) * ng + zg * hid_ref[...]

    return pl.pallas_call(
        body,
        out_shape=jax.ShapeDtypeStruct((R2, D), jnp.float32),
        grid=(R2 // TCB,),
        in_specs=[
            pl.BlockSpec((NC, TCB, D), lambda i: (0, i, 0)),
            pl.BlockSpec((NC, TCB, 1), lambda i: (0, i, 0)),
            pl.BlockSpec((TCB, D), lambda i: (i, 0)),
            pl.BlockSpec((TCB, D), lambda i: (i, 0)),
            pl.BlockSpec((D, 3 * D), lambda i: (0, 0)),
            pl.BlockSpec((D, 3 * D), lambda i: (0, 0)),
            pl.BlockSpec((D, 3 * D), lambda i: (0, 0)),
            pl.BlockSpec((1, 3 * D), lambda i: (0, 0)),
            pl.BlockSpec((1, 3 * D), lambda i: (0, 0)),
        ],
        out_specs=pl.BlockSpec((TCB, D), lambda i: (i, 0)),
    )(sums2, counts3, srel, hidden, wihs, wihd, whh, bih, bhh)


def _tc_layer0(agg2, deg3, cur, nw, lw):
    def body(a_ref, d_ref, cur_ref, nw_ref, lw_ref, o_ref):
        a = a_ref[0] + a_ref[1]
        d = jnp.maximum(d_ref[...], 1.0)
        o_ref[...] = (jnp.dot(a, nw_ref[...], preferred_element_type=jnp.float32) / d
                      + jnp.dot(cur_ref[...], lw_ref[...],
                                preferred_element_type=jnp.float32))

    return pl.pallas_call(
        body,
        out_shape=jax.ShapeDtypeStruct((N, D), jnp.float32),
        grid=(N // TCB,),
        in_specs=[
            pl.BlockSpec((NC, TCB, D), lambda i: (0, i, 0)),
            pl.BlockSpec((TCB, 1), lambda i: (i, 0)),
            pl.BlockSpec((TCB, D), lambda i: (i, 0)),
            pl.BlockSpec((D, D), lambda i: (0, 0)),
            pl.BlockSpec((D, D), lambda i: (0, 0)),
        ],
        out_specs=pl.BlockSpec((TCB, D), lambda i: (i, 0)),
    )(agg2, deg3, cur, nw, lw)


def _tc_layer1(agg2, deg3, cur, h, nw, lw, gw, gb):
    def body(a_ref, d_ref, cur_ref, h_ref, nw_ref, lw_ref, gw_ref, gb_ref, o_ref):
        a = a_ref[0] + a_ref[1]
        d = jnp.maximum(d_ref[...], 1.0)
        cur2 = (jnp.dot(a, nw_ref[...], preferred_element_type=jnp.float32) / d
                + jnp.dot(cur_ref[...], lw_ref[...],
                          preferred_element_type=jnp.float32))
        g = jax.nn.sigmoid(
            jnp.dot(h_ref[...], gw_ref[...], preferred_element_type=jnp.float32)
            + gb_ref[...])
        o_ref[...] = g * cur2 + (1.0 - g) * h_ref[...]

    return pl.pallas_call(
        body,
        out_shape=jax.ShapeDtypeStruct((N, D), jnp.float32),
        grid=(N // TCB,),
        in_specs=[
            pl.BlockSpec((NC, TCB, D), lambda i: (0, i, 0)),
            pl.BlockSpec((TCB, 1), lambda i: (i, 0)),
            pl.BlockSpec((TCB, D), lambda i: (i, 0)),
            pl.BlockSpec((TCB, D), lambda i: (i, 0)),
            pl.BlockSpec((D, D), lambda i: (0, 0)),
            pl.BlockSpec((D, D), lambda i: (0, 0)),
            pl.BlockSpec((D, D), lambda i: (0, 0)),
            pl.BlockSpec((1, D), lambda i: (0, 0)),
        ],
        out_specs=pl.BlockSpec((TCB, D), lambda i: (i, 0)),
    )(agg2, deg3, cur, h, nw, lw, gw, gb)


def kernel(edges, static_entity_embed, static_relation_embed, gate_weight,
           gate_bias, gru_w_ih, gru_w_hh, gru_b_ih, gru_b_hh, neigh_w, loop_w):
    et = edges.transpose(0, 2, 1)
    wihs = gru_w_ih[:, :D].T
    wihd = gru_w_ih[:, D:].T
    whh = gru_w_hh.T
    bih = gru_b_ih.reshape(1, 3 * D)
    bhh = gru_b_hh.reshape(1, 3 * D)
    gb = gate_bias.reshape(1, D)

    pad0 = jnp.zeros((EPAD - E,), jnp.int32)
    padd = jnp.full((EPAD - E,), DUMMY, jnp.int32)

    a1 = _make_a1()
    a2 = _make_a2()
    b = _make_b()

    h = _tc_norm(static_entity_embed)
    evolved = static_relation_embed
    outs = []
    for t in range(T):
        src, rel, dst = et[t, 0], et[t, 1], et[t, 2]
        ents = jnp.concatenate([src, dst]).reshape(NW, NCHA, CA)
        rels2 = jnp.concatenate([rel, rel]).reshape(NW, NCHA, CA)
        src_p = jnp.concatenate([src, pad0]).reshape(NW, NCHB, CB)
        rel_p = jnp.concatenate([rel, pad0]).reshape(NW, NCHB, CB)
        dst_p = jnp.concatenate([dst, padd]).reshape(NW, NCHB, CB)

        table, deg = a1(ents, rels2)
        deg3 = deg.reshape(NPAD, 1)
        sums2, counts2 = a2(ents, rels2, table, h)
        evolved = _tc_rel(sums2, counts2.reshape(NC, NPAD, 1),
                          static_relation_embed, evolved,
                          wihs, wihd, whh, bih, bhh)

        agg2 = b(src_p, rel_p, dst_p, h, evolved)
        cur1 = _tc_layer0(agg2, deg3, h, neigh_w[0], loop_w[0])

        agg2b = b(src_p, rel_p, dst_p, cur1, evolved)
        h = _tc_layer1(agg2b, deg3, cur1, h, neigh_w[1], loop_w[1],
                       gate_weight, gb)
        outs.append(h)

    return jnp.stack(outs, axis=0), evolved


# speedup vs baseline: 1.0101x; 1.0101x over previous
import functools

import jax
import jax.numpy as jnp
from jax import lax
from jax.experimental import pallas as pl
from jax.experimental.pallas import tpu as pltpu
from jax.experimental.pallas import tpu_sc as plsc

N = 10000
R2 = 10000
D = 128
E = 160000
T = 3
P = 2 * E
TBL = N * R2

NC = 2
NS = 16
NW = NC * NS

NPAD = 10240
DUMMY = 10000
RPT = NPAD // NS

CA = 80
PWA = P // NW
NCHA = PWA // CA
PHA = 40
PHASES_A = ((0, 40), (40, 40), (80, 40), (120, 5))

CB = 64
NCHB = 80
PHB = 40
EPAD = NW * NCHB * CB

TCB = 1000


@functools.lru_cache(maxsize=None)
def _mesh():
    return plsc.VectorSubcoreMesh(core_axis_name="c", subcore_axis_name="s")


def _wid():
    return lax.axis_index("c") * NS + lax.axis_index("s")


def _zero_fill_1d(zvec):
    n = zvec.shape[0]

    @pl.loop(0, n // 16)
    def _(i):
        zvec[pl.ds(i * 16, 16)] = jnp.zeros((16,), jnp.float32)


@functools.lru_cache(maxsize=None)
def _make_a1():
    @functools.partial(
        pl.kernel,
        out_type=(
            pltpu.HBM((TBL,), jnp.int32),
            pltpu.HBM((NPAD,), jnp.float32),
        ),
        mesh=_mesh(),
        scratch_types=[
            pltpu.VMEM((NCHA, CA), jnp.int32),
            pltpu.VMEM((NCHA, CA), jnp.int32),
            pltpu.VMEM((NCHA, CA), jnp.int32),
            pltpu.VMEM((NCHA, CA), jnp.int32),
            pltpu.VMEM((1, CA), jnp.float32),
            pltpu.VMEM((RPT,), jnp.float32),
            pltpu.VMEM_SHARED((NPAD,), jnp.float32),
            pltpu.SemaphoreType.DMA((2,)),
            pltpu.SemaphoreType.DMA((2,)),
            pltpu.SemaphoreType.DMA((2,)),
        ],
    )
    def a1(ents, rels, table, deg_out,
           entbig, relbig, pidbig, valbig, onesb, zvec, deg_sh,
           semi, sems, semd):
        cid = lax.axis_index("c")
        sid = lax.axis_index("s")
        wid = _wid()
        base = wid * PWA
        r0 = sid * RPT
        on_core1 = cid == 1

        pltpu.async_copy(ents.at[wid], entbig, semi.at[0])
        pltpu.async_copy(rels.at[wid], relbig, semi.at[1])

        _zero_fill_1d(zvec)
        for m in range(CA // 16):
            onesb[0, pl.ds(m * 16, 16)] = jnp.ones((16,), jnp.float32)

        @pl.when(on_core1)
        def _():
            pltpu.sync_copy(zvec, deg_sh.at[pl.ds(r0, RPT)])

        plsc.subcore_barrier()

        pltpu.make_async_copy(ents.at[wid], entbig, semi.at[0]).wait()
        pltpu.make_async_copy(rels.at[wid], relbig, semi.at[1]).wait()

        def step(j, b):
            for m in range(CA // 16):
                sl = pl.ds(m * 16, 16)
                pidbig[j, sl] = entbig[j, sl] * R2 + relbig[j, sl]
                valbig[j, sl] = (base + j * CA + m * 16) + lax.iota(jnp.int32, 16)
            pltpu.async_copy(valbig.at[j], table.at[pidbig.at[j]], sems.at[b])

            @pl.when(on_core1)
            def _():
                pltpu.async_copy(onesb.at[0], deg_sh.at[entbig.at[j]],
                                 semd.at[b], add=True)

        @pl.loop(0, NCHA - 1, step=2)
        def _(j0):
            step(j0, 0)
            step(j0 + 1, 1)

        step(NCHA - 1, 0)

        @pl.loop(0, (NCHA + 1) // 2)
        def _(j):
            pltpu.make_async_copy(valbig.at[0], table.at[pidbig.at[0]], sems.at[0]).wait()

        @pl.loop(0, NCHA // 2)
        def _(j):
            pltpu.make_async_copy(valbig.at[0], table.at[pidbig.at[0]], sems.at[1]).wait()

        @pl.when(on_core1)
        def _():
            @pl.loop(0, (NCHA + 1) // 2)
            def _(j):
                pltpu.make_async_copy(onesb.at[0], deg_sh.at[entbig.at[0]], semd.at[0]).wait()

            @pl.loop(0, NCHA // 2)
            def _(j):
                pltpu.make_async_copy(onesb.at[0], deg_sh.at[entbig.at[0]], semd.at[1]).wait()

            plsc.subcore_barrier()
            pltpu.sync_copy(deg_sh.at[pl.ds(r0, RPT)], deg_out.at[pl.ds(r0, RPT)])

    return a1


@functools.lru_cache(maxsize=None)
def _make_a2():
    @functools.partial(
        pl.kernel,
        out_type=(
            pltpu.HBM((NC, NPAD, D), jnp.float32),
            pltpu.HBM((NC, NPAD), jnp.float32),
        ),
        mesh=_mesh(),
        scratch_types=[
            pltpu.VMEM((PHA, CA), jnp.int32),
            pltpu.VMEM((PHA, CA), jnp.int32),
            pltpu.VMEM((2, CA), jnp.int32),
            pltpu.VMEM((2, CA), jnp.int32),
            pltpu.VMEM((2, CA), jnp.int32),
            pltpu.VMEM((2, CA, D), jnp.float32),
            pltpu.VMEM((1, CA), jnp.float32),
            pltpu.VMEM((RPT,), jnp.float32),
            pltpu.VMEM_SHARED((NPAD, D), jnp.float32),
            pltpu.VMEM_SHARED((NPAD,), jnp.float32),
            pltpu.SemaphoreType.DMA((2,)),
            pltpu.SemaphoreType.DMA((2,)),
            pltpu.SemaphoreType.DMA((2,)),
        ],
    )
    def a2(ents, rels, table, h, sums_out, cnt_out,
           entbig, relbig, pidb, tvb, selb, rowsb, onesb, zvec,
           sums_sh, cnt_sh, semi, semg, semsc):
        cid = lax.axis_index("c")
        sid = lax.axis_index("s")
        wid = _wid()
        base = wid * PWA
        r0 = sid * RPT

        def load_idx(p0, ph):
            pltpu.async_copy(ents.at[wid, pl.ds(p0, ph)], entbig.at[pl.ds(0, ph)], semi.at[0])
            pltpu.async_copy(rels.at[wid, pl.ds(p0, ph)], relbig.at[pl.ds(0, ph)], semi.at[1])

        def wait_idx(p0, ph):
            pltpu.make_async_copy(ents.at[wid, pl.ds(p0, ph)], entbig.at[pl.ds(0, ph)], semi.at[0]).wait()
            pltpu.make_async_copy(rels.at[wid, pl.ds(p0, ph)], relbig.at[pl.ds(0, ph)], semi.at[1]).wait()

        load_idx(0, PHA)

        @pl.loop(0, CA)
        def _(i):
            for k in range(D // 16):
                rowsb[0, i, pl.ds(k * 16, 16)] = jnp.zeros((16,), jnp.float32)

        _zero_fill_1d(zvec)
        for m in range(CA // 16):
            onesb[0, pl.ds(m * 16, 16)] = jnp.ones((16,), jnp.float32)

        @pl.loop(0, RPT // CA)
        def _(jj):
            pltpu.sync_copy(rowsb.at[0], sums_sh.at[pl.ds(r0 + jj * CA, CA)])

        pltpu.sync_copy(zvec, cnt_sh.at[pl.ds(r0, RPT)])
        plsc.subcore_barrier()

        def fire_gather(i, sl):
            for m in range(CA // 16):
                slx = pl.ds(m * 16, 16)
                pidb[sl, slx] = entbig[i, slx] * R2 + relbig[i, slx]
            pltpu.async_copy(table.at[pidb.at[sl]], tvb.at[sl], semg.at[sl])
            pltpu.async_copy(h.at[entbig.at[i]], rowsb.at[sl], semg.at[sl])

        def wait_scatters(sl):
            pltpu.make_async_copy(rowsb.at[sl], sums_sh.at[selb.at[sl]], semsc.at[sl]).wait()
            pltpu.make_async_copy(onesb.at[0], cnt_sh.at[selb.at[sl]], semsc.at[sl]).wait()

        def consume(i, p0, sl):
            pltpu.make_async_copy(table.at[pidb.at[sl]], tvb.at[sl], semg.at[sl]).wait()
            pltpu.make_async_copy(h.at[entbig.at[0]], rowsb.at[sl], semg.at[sl]).wait()
            for m in range(CA // 16):
                slx = pl.ds(m * 16, 16)
                val16 = (base + p0 * CA + i * CA + m * 16) + lax.iota(jnp.int32, 16)
                first = tvb[sl, slx] == val16
                selb[sl, slx] = jnp.where(first, relbig[i, slx], DUMMY)
            pltpu.async_copy(rowsb.at[sl], sums_sh.at[selb.at[sl]], semsc.at[sl], add=True)
            pltpu.async_copy(onesb.at[0], cnt_sh.at[selb.at[sl]], semsc.at[sl], add=True)

        for pi, (p0, ph) in enumerate(PHASES_A):
            if pi > 0:
                prev_ph = PHASES_A[pi - 1][1]
                wait_scatters((prev_ph - 1) % 2)
                load_idx(p0, ph)
                wait_idx(p0, ph)
            else:
                wait_idx(0, PHA)
            fire_gather(0, 0)
            if ph % 2 == 0:
                @pl.loop(0, ph - 2, step=2)
                def _(j0, p0=p0):
                    @pl.when(j0 >= 1)
                    def _():
                        wait_scatters(1)

                    fire_gather(j0 + 1, 1)
                    consume(j0, p0, 0)
                    wait_scatters(0)
                    fire_gather(j0 + 2, 0)
                    consume(j0 + 1, p0, 1)

                if ph > 2:
                    wait_scatters(1)
                fire_gather(ph - 1, 1)
                consume(ph - 2, p0, 0)
                wait_scatters(0)
                consume(ph - 1, p0, 1)
            else:
                @pl.loop(0, ph - 1, step=2)
                def _(j0, p0=p0):
                    @pl.when(j0 >= 1)
                    def _():
                        wait_scatters(1)

                    fire_gather(j0 + 1, 1)
                    consume(j0, p0, 0)
                    wait_scatters(0)
                    fire_gather(j0 + 2, 0)
                    consume(j0 + 1, p0, 1)

                if ph > 2:
                    wait_scatters(1)
                consume(ph - 1, p0, 0)

        wait_scatters((PHASES_A[-1][1] - 1) % 2)
        plsc.subcore_barrier()

        @pl.loop(0, RPT // 128)
        def _(jj):
            pltpu.sync_copy(sums_sh.at[pl.ds(r0 + jj * 128, 128)],
                            sums_out.at[cid, pl.ds(r0 + jj * 128, 128)])

        pltpu.sync_copy(cnt_sh.at[pl.ds(r0, RPT)], cnt_out.at[cid, pl.ds(r0, RPT)])

    return a2


@functools.lru_cache(maxsize=None)
def _make_b():
    @functools.partial(
        pl.kernel,
        out_type=pltpu.HBM((NC, NPAD, D), jnp.float32),
        mesh=_mesh(),
        scratch_types=[
            pltpu.VMEM((PHB, CB), jnp.int32),
            pltpu.VMEM((PHB, CB), jnp.int32),
            pltpu.VMEM((PHB, CB), jnp.int32),
            pltpu.VMEM((2, CB, D), jnp.float32),
            pltpu.VMEM((2, CB, D), jnp.float32),
            pltpu.VMEM_SHARED((NPAD, D), jnp.float32),
            pltpu.SemaphoreType.DMA((3,)),
            pltpu.SemaphoreType.DMA((2,)),
            pltpu.SemaphoreType.DMA((2,)),
        ],
    )
    def b(src, rel, dst, taba, tabb, agg_out,
          sb, rb, db, rowsa, rowsb, agg_sh, semi, semg, semsc):
        cid = lax.axis_index("c")
        sid = lax.axis_index("s")
        wid = _wid()
        r0 = sid * RPT

        def load_idx(p0):
            pltpu.async_copy(src.at[wid, pl.ds(p0, PHB)], sb, semi.at[0])
            pltpu.async_copy(rel.at[wid, pl.ds(p0, PHB)], rb, semi.at[1])
            pltpu.async_copy(dst.at[wid, pl.ds(p0, PHB)], db, semi.at[2])

        def wait_idx(p0):
            pltpu.make_async_copy(src.at[wid, pl.ds(p0, PHB)], sb, semi.at[0]).wait()
            pltpu.make_async_copy(rel.at[wid, pl.ds(p0, PHB)], rb, semi.at[1]).wait()
            pltpu.make_async_copy(dst.at[wid, pl.ds(p0, PHB)], db, semi.at[2]).wait()

        load_idx(0)

        @pl.loop(0, CB)
        def _(i):
            for k in range(D // 16):
                rowsa[0, i, pl.ds(k * 16, 16)] = jnp.zeros((16,), jnp.float32)

        @pl.loop(0, RPT // CB)
        def _(jj):
            pltpu.sync_copy(rowsa.at[0], agg_sh.at[pl.ds(r0 + jj * CB, CB)])

        plsc.subcore_barrier()

        def fire_gather(i, sl):
            pltpu.async_copy(taba.at[sb.at[i]], rowsa.at[sl], semg.at[sl])
            pltpu.async_copy(tabb.at[rb.at[i]], rowsb.at[sl], semg.at[sl])

        def wait_scatters(sl):
            pltpu.make_async_copy(rowsa.at[sl], agg_sh.at[db.at[0]], semsc.at[sl]).wait()
            pltpu.make_async_copy(rowsb.at[sl], agg_sh.at[db.at[0]], semsc.at[sl]).wait()

        def consume(i, sl):
            pltpu.make_async_copy(taba.at[sb.at[0]], rowsa.at[sl], semg.at[sl]).wait()
            pltpu.make_async_copy(tabb.at[rb.at[0]], rowsb.at[sl], semg.at[sl]).wait()
            pltpu.async_copy(rowsa.at[sl], agg_sh.at[db.at[i]], semsc.at[sl], add=True)
            pltpu.async_copy(rowsb.at[sl], agg_sh.at[db.at[i]], semsc.at[sl], add=True)

        for ph in range(NCHB // PHB):
            if ph > 0:
                wait_scatters(1)
                load_idx(ph * PHB)
                wait_idx(ph * PHB)
            else:
                wait_idx(0)
            fire_gather(0, 0)

            @pl.loop(0, PHB - 2, step=2)
            def _(j0):
                @pl.when(j0 >= 1)
                def _():
                    wait_scatters(1)

                fire_gather(j0 + 1, 1)
                consume(j0, 0)
                wait_scatters(0)
                fire_gather(j0 + 2, 0)
                consume(j0 + 1, 1)

            wait_scatters(1)
            fire_gather(PHB - 1, 1)
            consume(PHB - 2, 0)
            wait_scatters(0)
            consume(PHB - 1, 1)

        wait_scatters(1)
        plsc.subcore_barrier()

        @pl.loop(0, RPT // 128)
        def _(jj):
            pltpu.sync_copy(agg_sh.at[pl.ds(r0 + jj * 128, 128)],
                            agg_out.at[cid, pl.ds(r0 + jj * 128, 128)])

    return b


def _tc_norm(x):
    def body(x_ref, o_ref):
        v = x_ref[...]
        n = jnp.sqrt(jnp.sum(v * v, axis=1, keepdims=True))
        o_ref[...] = v / jnp.maximum(n, 1e-12)

    return pl.pallas_call(
        body,
        out_shape=jax.ShapeDtypeStruct((N, D), jnp.float32),
        grid=(N // TCB,),
        in_specs=[pl.BlockSpec((TCB, D), lambda i: (i, 0))],
        out_specs=pl.BlockSpec((TCB, D), lambda i: (i, 0)),
    )(x)


def _tc_rel(sums2, counts3, srel, hidden, wihs, wihd, whh, bih, bhh):
    def body(s_ref, c_ref, sr_ref, hid_ref, wihs_ref, wihd_ref, whh_ref,
             bih_ref, bhh_ref, o_ref):
        s = s_ref[0] + s_ref[1]
        c = c_ref[0] + c_ref[1]
        dyn = jnp.where(c > 0.0, s / jnp.maximum(c, 1.0), 0.0)
        gi = (jnp.dot(sr_ref[...], wihs_ref[...], preferred_element_type=jnp.float32)
              + jnp.dot(dyn, wihd_ref[...], preferred_element_type=jnp.float32)
              + bih_ref[...])
        gh = (jnp.dot(hid_ref[...], whh_ref[...], preferred_element_type=jnp.float32)
              + bhh_ref[...])
        rg = jax.nn.sigmoid(gi[:, :D] + gh[:, :D])
        zg = jax.nn.sigmoid(gi[:, D:2 * D] + gh[:, D:2 * D])
        ng = jnp.tanh(gi[:, 2 * D:] + rg * gh[:, 2 * D:])
        o_ref[...] = (1.0 - zg) * ng + zg * hid_ref[...]

    return pl.pallas_call(
        body,
        out_shape=jax.ShapeDtypeStruct((R2, D), jnp.float32),
        grid=(R2 // TCB,),
        in_specs=[
            pl.BlockSpec((NC, TCB, D), lambda i: (0, i, 0)),
            pl.BlockSpec((NC, TCB, 1), lambda i: (0, i, 0)),
            pl.BlockSpec((TCB, D), lambda i: (i, 0)),
            pl.BlockSpec((TCB, D), lambda i: (i, 0)),
            pl.BlockSpec((D, 3 * D), lambda i: (0, 0)),
            pl.BlockSpec((D, 3 * D), lambda i: (0, 0)),
            pl.BlockSpec((D, 3 * D), lambda i: (0, 0)),
            pl.BlockSpec((1, 3 * D), lambda i: (0, 0)),
            pl.BlockSpec((1, 3 * D), lambda i: (0, 0)),
        ],
        out_specs=pl.BlockSpec((TCB, D), lambda i: (i, 0)),
    )(sums2, counts3, srel, hidden, wihs, wihd, whh, bih, bhh)


def _tc_layer0(agg2, deg3, cur, nw, lw):
    def body(a_ref, d_ref, cur_ref, nw_ref, lw_ref, o_ref):
        a = a_ref[0] + a_ref[1]
        d = jnp.maximum(d_ref[...], 1.0)
        o_ref[...] = (jnp.dot(a, nw_ref[...], preferred_element_type=jnp.float32) / d
                      + jnp.dot(cur_ref[...], lw_ref[...],
                                preferred_element_type=jnp.float32))

    return pl.pallas_call(
        body,
        out_shape=jax.ShapeDtypeStruct((N, D), jnp.float32),
        grid=(N // TCB,),
        in_specs=[
            pl.BlockSpec((NC, TCB, D), lambda i: (0, i, 0)),
            pl.BlockSpec((TCB, 1), lambda i: (i, 0)),
            pl.BlockSpec((TCB, D), lambda i: (i, 0)),
            pl.BlockSpec((D, D), lambda i: (0, 0)),
            pl.BlockSpec((D, D), lambda i: (0, 0)),
        ],
        out_specs=pl.BlockSpec((TCB, D), lambda i: (i, 0)),
    )(agg2, deg3, cur, nw, lw)


def _tc_layer1(agg2, deg3, cur, h, nw, lw, gw, gb):
    def body(a_ref, d_ref, cur_ref, h_ref, nw_ref, lw_ref, gw_ref, gb_ref, o_ref):
        a = a_ref[0] + a_ref[1]
        d = jnp.maximum(d_ref[...], 1.0)
        cur2 = (jnp.dot(a, nw_ref[...], preferred_element_type=jnp.float32) / d
                + jnp.dot(cur_ref[...], lw_ref[...],
                          preferred_element_type=jnp.float32))
        g = jax.nn.sigmoid(
            jnp.dot(h_ref[...], gw_ref[...], preferred_element_type=jnp.float32)
            + gb_ref[...])
        o_ref[...] = g * cur2 + (1.0 - g) * h_ref[...]

    return pl.pallas_call(
        body,
        out_shape=jax.ShapeDtypeStruct((N, D), jnp.float32),
        grid=(N // TCB,),
        in_specs=[
            pl.BlockSpec((NC, TCB, D), lambda i: (0, i, 0)),
            pl.BlockSpec((TCB, 1), lambda i: (i, 0)),
            pl.BlockSpec((TCB, D), lambda i: (i, 0)),
            pl.BlockSpec((TCB, D), lambda i: (i, 0)),
            pl.BlockSpec((D, D), lambda i: (0, 0)),
            pl.BlockSpec((D, D), lambda i: (0, 0)),
            pl.BlockSpec((D, D), lambda i: (0, 0)),
            pl.BlockSpec((1, D), lambda i: (0, 0)),
        ],
        out_specs=pl.BlockSpec((TCB, D), lambda i: (i, 0)),
    )(agg2, deg3, cur, h, nw, lw, gw, gb)


def kernel(edges, static_entity_embed, static_relation_embed, gate_weight,
           gate_bias, gru_w_ih, gru_w_hh, gru_b_ih, gru_b_hh, neigh_w, loop_w):
    et = edges.transpose(0, 2, 1)
    wihs = gru_w_ih[:, :D].T
    wihd = gru_w_ih[:, D:].T
    whh = gru_w_hh.T
    bih = gru_b_ih.reshape(1, 3 * D)
    bhh = gru_b_hh.reshape(1, 3 * D)
    gb = gate_bias.reshape(1, D)

    pad0 = jnp.zeros((EPAD - E,), jnp.int32)
    padd = jnp.full((EPAD - E,), DUMMY, jnp.int32)

    a1 = _make_a1()
    a2 = _make_a2()
    b = _make_b()

    h = _tc_norm(static_entity_embed)
    evolved = static_relation_embed
    outs = []
    for t in range(T):
        src, rel, dst = et[t, 0], et[t, 1], et[t, 2]
        ents = jnp.concatenate([src, dst]).reshape(NW, NCHA, CA)
        rels2 = jnp.concatenate([rel, rel]).reshape(NW, NCHA, CA)
        src_p = jnp.concatenate([src, pad0]).reshape(NW, NCHB, CB)
        rel_p = jnp.concatenate([rel, pad0]).reshape(NW, NCHB, CB)
        dst_p = jnp.concatenate([dst, padd]).reshape(NW, NCHB, CB)

        table, deg = a1(ents, rels2)
        deg3 = deg.reshape(NPAD, 1)
        sums2, counts2 = a2(ents, rels2, table, h)
        evolved = _tc_rel(sums2, counts2.reshape(NC, NPAD, 1),
                          static_relation_embed, evolved,
                          wihs, wihd, whh, bih, bhh)

        agg2 = b(src_p, rel_p, dst_p, h, evolved)
        cur1 = _tc_layer0(agg2, deg3, h, neigh_w[0], loop_w[0])

        agg2b = b(src_p, rel_p, dst_p, cur1, evolved)
        h = _tc_layer1(agg2b, deg3, cur1, h, neigh_w[1], loop_w[1],
                       gate_weight, gb)
        outs.append(h)

    return jnp.stack(outs, axis=0), evolved
